# baseline (speedup 1.0000x reference)
import jax
import jax.numpy as jnp
from jax.experimental import pallas as pl
from jax.experimental.pallas import tpu as pltpu

BT = 256


def _enc_expert_kernel(xb, We1, We2, We3, Wx1, Wx2, be1, be2, be3, bx1, bx2,
                       lat_out, log_out):
    f32 = jnp.float32
    xbf = xb[...].astype(jnp.bfloat16)
    h = jnp.dot(xbf, We1[...], preferred_element_type=f32) + be1[...]
    h = jnp.maximum(h, 0.0).astype(jnp.bfloat16)
    h = jnp.dot(h, We2[...], preferred_element_type=f32) + be2[...]
    h = jnp.maximum(h, 0.0).astype(jnp.bfloat16)
    lat = jnp.dot(h, We3[...], preferred_element_type=f32) + be3[...]
    latb = lat.astype(jnp.bfloat16)
    lat_out[...] = latb
    eh = jnp.dot(latb, Wx1[...], preferred_element_type=f32) + bx1[...]
    eh = jnp.maximum(eh, 0.0).astype(jnp.bfloat16)
    log_out[...] = jnp.dot(eh, Wx2[...], preferred_element_type=f32) + bx2[...]


def _dec_err_kernel(latb, xb, Wd1, Wd2, Wd3, bd1, bd2, bd3, err_out):
    f32 = jnp.float32
    d = jnp.dot(latb[...], Wd1[...], preferred_element_type=f32) + bd1[...]
    d = jnp.maximum(d, 0.0).astype(jnp.bfloat16)
    d = jnp.dot(d, Wd2[...], preferred_element_type=f32) + bd2[...]
    d = jnp.maximum(d, 0.0).astype(jnp.bfloat16)
    recon = jnp.dot(d, Wd3[...], preferred_element_type=f32) + bd3[...]
    err = jnp.sum(jnp.abs(recon - xb[...]), axis=1) / recon.shape[1]
    err_out[...] = err


def _full(shape):
    nd = len(shape)
    return pl.BlockSpec(shape, lambda i: (0,) * nd)


def kernel(x, We1, be1, We2, be2, We3, be3, Wd1, bd1, Wd2, bd2, Wd3, bd3,
           Wx1, bx1, Wx2, bx2):
    B = x.shape[0]
    FLAT = x.shape[1] * x.shape[2] * x.shape[3]
    HIDDEN = We1.shape[1]
    LATENT = We3.shape[1]
    CLASSES = Wx2.shape[1]
    NPAD = 128

    flat = x.reshape(B, FLAT)
    bf = jnp.bfloat16
    We1b, We2b, We3b = We1.astype(bf), We2.astype(bf), We3.astype(bf)
    Wd1b, Wd2b, Wd3b = Wd1.astype(bf), Wd2.astype(bf), Wd3.astype(bf)
    Wx1b = Wx1.astype(bf)
    Wx2b = jnp.zeros((HIDDEN, NPAD), bf).at[:, :CLASSES].set(Wx2.astype(bf))
    bx2p = jnp.zeros((1, NPAD), jnp.float32).at[:, :CLASSES].set(bx2)
    be1r, be2r, be3r = be1[None, :], be2[None, :], be3[None, :]
    bd1r, bd2r, bd3r = bd1[None, :], bd2[None, :], bd3[None, :]
    bx1r = bx1[None, :]

    nsteps = B // BT
    bspec = lambda n: pl.BlockSpec((BT, n), lambda i: (i, 0))

    lat_b, log_pad = pl.pallas_call(
        _enc_expert_kernel,
        grid=(nsteps,),
        in_specs=[
            bspec(FLAT),
            _full((FLAT, HIDDEN)), _full((HIDDEN, HIDDEN)),
            _full((HIDDEN, LATENT)), _full((LATENT, HIDDEN)),
            _full((HIDDEN, NPAD)),
            _full((1, HIDDEN)), _full((1, HIDDEN)), _full((1, LATENT)),
            _full((1, HIDDEN)), _full((1, NPAD)),
        ],
        out_specs=[bspec(LATENT), bspec(NPAD)],
        out_shape=[
            jax.ShapeDtypeStruct((B, LATENT), bf),
            jax.ShapeDtypeStruct((B, NPAD), jnp.float32),
        ],
        compiler_params=pltpu.CompilerParams(
            dimension_semantics=("arbitrary",),
            vmem_limit_bytes=60 * 1024 * 1024,
        ),
    )(flat, We1b, We2b, We3b, Wx1b, Wx2b, be1r, be2r, be3r, bx1r, bx2p)

    err = pl.pallas_call(
        _dec_err_kernel,
        grid=(nsteps,),
        in_specs=[
            bspec(LATENT), bspec(FLAT),
            _full((LATENT, HIDDEN)), _full((HIDDEN, HIDDEN)),
            _full((HIDDEN, FLAT)),
            _full((1, HIDDEN)), _full((1, HIDDEN)), _full((1, FLAT)),
        ],
        out_specs=pl.BlockSpec((BT,), lambda i: (i,)),
        out_shape=jax.ShapeDtypeStruct((B,), jnp.float32),
        compiler_params=pltpu.CompilerParams(
            dimension_semantics=("arbitrary",),
            vmem_limit_bytes=60 * 1024 * 1024,
        ),
    )(lat_b, flat, Wd1b, Wd2b, Wd3b, bd1r, bd2r, bd3r)

    logits = log_pad[:, :CLASSES]
    indices = jnp.zeros((B,), jnp.int32)
    min_err = err
    relevance_scores = jnp.ones((1, B), jnp.float32)
    mask = jnp.ones((1, B), jnp.bool_)
    return (logits, indices, min_err, relevance_scores, mask)
